# baseline (speedup 1.0000x reference)
import functools

import jax
import jax.numpy as jnp
from jax import lax
from jax.experimental import pallas as pl
from jax.experimental.pallas import tpu as pltpu
from jax.experimental.pallas import tpu_sc as plsc


def _node_msg_body(x_ref, w1_ref, b1_ref, w2_ref, b2_ref, o_ref):
    x = x_ref[...]
    nrm2 = jnp.sum(x * x, axis=1, keepdims=True)
    nrm = jnp.maximum(jnp.sqrt(nrm2), 1e-15)
    s = jnp.minimum(nrm, 1.0 - 1e-7)
    atanh_s = 0.5 * jnp.log((1.0 + s) / (1.0 - s))
    t = x * (atanh_s / nrm)
    h = jnp.maximum(
        jnp.dot(t, w1_ref[...], preferred_element_type=jnp.float32) + b1_ref[...], 0.0
    )
    o_ref[...] = (
        jnp.dot(h, w2_ref[...], preferred_element_type=jnp.float32) + b2_ref[...]
    )


def _node_messages(x_pad, W1, b1, W2, b2, blk):
    np_, d = x_pad.shape
    grid = (np_ // blk,)
    return pl.pallas_call(
        _node_msg_body,
        grid=grid,
        in_specs=[
            pl.BlockSpec((blk, d), lambda i: (i, 0)),
            pl.BlockSpec((d, d), lambda i: (0, 0)),
            pl.BlockSpec((1, d), lambda i: (0, 0)),
            pl.BlockSpec((d, d), lambda i: (0, 0)),
            pl.BlockSpec((1, d), lambda i: (0, 0)),
        ],
        out_specs=pl.BlockSpec((blk, d), lambda i: (i, 0)),
        out_shape=jax.ShapeDtypeStruct((np_, d), jnp.float32),
    )(x_pad, W1, b1.reshape(1, d), W2, b2.reshape(1, d))


def _update_body(acc_ref, cnt_ref, u1_ref, ub1_ref, u2_ref, ub2_ref, o_ref, *, blk):
    acc = acc_ref[...]
    s = acc[0] + acc[1]
    cnt = cnt_ref[...]
    c = jnp.sum(cnt, axis=0)
    inv = 1.0 / (c + 1e-08)
    mean = (s.reshape(blk // 128, 128, 128) * inv[:, :, None]).reshape(blk, 128)
    h = jnp.maximum(
        jnp.dot(mean, u1_ref[...], preferred_element_type=jnp.float32) + ub1_ref[...],
        0.0,
    )
    o = jnp.dot(h, u2_ref[...], preferred_element_type=jnp.float32) + ub2_ref[...]
    nrm = jnp.maximum(jnp.sqrt(jnp.sum(o * o, axis=1, keepdims=True)), 1e-15)
    o_ref[...] = o * (jnp.tanh(nrm) / nrm)


def _update(acc, cnt2, U1, ub1, U2, ub2, blk):
    _, np_, d = acc.shape
    nw = cnt2.shape[0]
    grid = (np_ // blk,)
    return pl.pallas_call(
        functools.partial(_update_body, blk=blk),
        grid=grid,
        in_specs=[
            pl.BlockSpec((2, blk, d), lambda i: (0, i, 0)),
            pl.BlockSpec((nw, blk // 128, 128), lambda i: (0, i, 0)),
            pl.BlockSpec((d, d), lambda i: (0, 0)),
            pl.BlockSpec((1, d), lambda i: (0, 0)),
            pl.BlockSpec((d, d), lambda i: (0, 0)),
            pl.BlockSpec((1, d), lambda i: (0, 0)),
        ],
        out_specs=pl.BlockSpec((blk, d), lambda i: (i, 0)),
        out_shape=jax.ShapeDtypeStruct((np_, d), jnp.float32),
    )(acc, cnt2, U1, ub1.reshape(1, d), U2, ub2.reshape(1, d))


def _make_sc_scatter(np_, d, nw, nc, ns, ch, b):
    rows_per_tile = np_ // ns

    mesh = plsc.VectorSubcoreMesh(core_axis_name="c", subcore_axis_name="s")

    @functools.partial(
        pl.kernel,
        mesh=mesh,
        out_type=[
            jax.ShapeDtypeStruct((nc, np_, d), jnp.float32),
            jax.ShapeDtypeStruct((nc, np_), jnp.float32),
        ],
        scratch_types=[
            pltpu.VMEM((ch // 2, b), jnp.int32),
            pltpu.VMEM((ch // 2, b), jnp.int32),
            pltpu.VMEM((2, b, d), jnp.float32),
            pltpu.VMEM((b,), jnp.float32),
            pltpu.VMEM_SHARED((np_, d), jnp.float32),
            pltpu.VMEM_SHARED((np_,), jnp.float32),
            pltpu.SemaphoreType.DMA,
            pltpu.SemaphoreType.DMA,
            pltpu.SemaphoreType.DMA,
            pltpu.SemaphoreType.DMA,
        ],
    )
    def sc_kernel(
        m_hbm,
        row_hbm,
        col_hbm,
        acc_hbm,
        cnt_hbm,
        row_v,
        col_v,
        rows_v,
        ones_v,
        acc_sh,
        cnt_sh,
        gsem,
        csem,
        ssem0,
        ssem1,
    ):
        cid = lax.axis_index("c")
        sid = lax.axis_index("s")
        wid = sid * nc + cid

        def _fill(i, _):
            for j in range(d // 16):
                rows_v[0, i, pl.ds(j * 16, 16)] = jnp.zeros((16,), jnp.float32)
            return 0

        lax.fori_loop(0, b, _fill, 0)
        for j in range(b // 16):
            ones_v[pl.ds(j * 16, 16)] = jnp.ones((16,), jnp.float32)

        base = sid * rows_per_tile
        for k in range(rows_per_tile // b):
            pltpu.sync_copy(rows_v.at[0], acc_sh.at[pl.ds(base + k * b, b)])
        for k in range(rows_per_tile // d):
            pltpu.sync_copy(rows_v.at[0, 0], cnt_sh.at[pl.ds(base + k * d, d)])
        plsc.subcore_barrier()

        ch2 = ch // 2
        ssems = (ssem0, ssem1)

        def _drain_scatter(bi):
            pltpu.make_async_copy(
                m_hbm.at[pl.ds(0, b)], rows_v.at[bi], ssems[bi]
            ).wait()

        for half in range(2):
            pltpu.sync_copy(row_hbm.at[wid, half], row_v)
            pltpu.sync_copy(col_hbm.at[wid, half], col_v)

            def _pair(q, _):
                c0 = q * 2
                for bi in range(2):
                    c = c0 + bi
                    pltpu.async_copy(m_hbm.at[col_v.at[c]], rows_v.at[bi], gsem).wait()
                    pltpu.async_copy(ones_v, cnt_sh.at[row_v.at[c]], csem, add=True)
                    pltpu.async_copy(
                        rows_v.at[bi], acc_sh.at[row_v.at[c]], ssems[bi], add=True
                    )
                return 0

            def _pair_steady(q, _):
                c0 = q * 2
                for bi in range(2):
                    c = c0 + bi
                    _drain_scatter(bi)
                    pltpu.async_copy(m_hbm.at[col_v.at[c]], rows_v.at[bi], gsem).wait()
                    pltpu.async_copy(ones_v, cnt_sh.at[row_v.at[c]], csem, add=True)
                    pltpu.async_copy(
                        rows_v.at[bi], acc_sh.at[row_v.at[c]], ssems[bi], add=True
                    )
                return 0

            _pair(0, 0)
            lax.fori_loop(1, ch2 // 2, _pair_steady, 0)
            for bi in range(2):
                _drain_scatter(bi)
            pltpu.make_async_copy(row_hbm.at[wid, half], row_v, csem).wait()
        plsc.subcore_barrier()

        pltpu.sync_copy(
            acc_sh.at[pl.ds(base, rows_per_tile)],
            acc_hbm.at[cid, pl.ds(base, rows_per_tile)],
        )
        pltpu.sync_copy(
            cnt_sh.at[pl.ds(base, rows_per_tile)],
            cnt_hbm.at[cid, pl.ds(base, rows_per_tile)],
        )

    return sc_kernel


def kernel(x, edge_index, W1, b1, W2, b2, U1, ub1, U2, ub2):
    n, d = x.shape
    e = edge_index.shape[1]

    info = plsc.get_sparse_core_info()
    nc, ns = info.num_cores, info.num_subcores
    nw = nc * ns
    b = 128
    quantum = nw * b * 4
    ep = ((e + quantum - 1) // quantum) * quantum
    ch = ep // (nw * b)

    blk = 1024
    np_ = ((n + blk - 1) // blk) * blk

    x_pad = jnp.pad(x, ((0, np_ - n), (0, 0)))
    m = _node_messages(x_pad, W1, b1, W2, b2, blk)

    row = edge_index[0]
    col = edge_index[1]
    pad_e = ep - e
    row_p = jnp.concatenate([row, jnp.full((pad_e,), n, jnp.int32)]).reshape(
        nw, 2, ch // 2, b
    )
    col_p = jnp.concatenate([col, jnp.zeros((pad_e,), jnp.int32)]).reshape(
        nw, 2, ch // 2, b
    )

    acc, cnt = _make_sc_scatter(np_, d, nw, nc, ns, ch, b)(m, row_p, col_p)

    cnt2 = cnt.reshape(nc, np_ // 128, 128)
    out = _update(acc, cnt2, U1, ub1, U2, ub2, blk)
    return out[:n]


# speedup vs baseline: 1.2231x; 1.2231x over previous
import functools

import jax
import jax.numpy as jnp
from jax import lax
from jax.experimental import pallas as pl
from jax.experimental.pallas import tpu as pltpu
from jax.experimental.pallas import tpu_sc as plsc


def _node_msg_body(x_ref, w1_ref, b1_ref, w2_ref, b2_ref, o_ref):
    x = x_ref[...]
    nrm2 = jnp.sum(x * x, axis=1, keepdims=True)
    nrm = jnp.maximum(jnp.sqrt(nrm2), 1e-15)
    s = jnp.minimum(nrm, 1.0 - 1e-7)
    atanh_s = 0.5 * jnp.log((1.0 + s) / (1.0 - s))
    t = x * (atanh_s / nrm)
    h = jnp.maximum(
        jnp.dot(t, w1_ref[...], preferred_element_type=jnp.float32) + b1_ref[...], 0.0
    )
    o_ref[...] = (
        jnp.dot(h, w2_ref[...], preferred_element_type=jnp.float32) + b2_ref[...]
    )


def _node_messages(x_pad, W1, b1, W2, b2, blk):
    np_, d = x_pad.shape
    grid = (np_ // blk,)
    return pl.pallas_call(
        _node_msg_body,
        grid=grid,
        in_specs=[
            pl.BlockSpec((blk, d), lambda i: (i, 0)),
            pl.BlockSpec((d, d), lambda i: (0, 0)),
            pl.BlockSpec((1, d), lambda i: (0, 0)),
            pl.BlockSpec((d, d), lambda i: (0, 0)),
            pl.BlockSpec((1, d), lambda i: (0, 0)),
        ],
        out_specs=pl.BlockSpec((blk, d), lambda i: (i, 0)),
        out_shape=jax.ShapeDtypeStruct((np_, d), jnp.float32),
    )(x_pad, W1, b1.reshape(1, d), W2, b2.reshape(1, d))


def _update_body(acc_ref, cnt_ref, u1_ref, ub1_ref, u2_ref, ub2_ref, o_ref, *, blk):
    acc = acc_ref[...]
    s = acc[0] + acc[1]
    cnt = cnt_ref[...]
    c = jnp.sum(cnt, axis=0)
    inv = 1.0 / (c + 1e-08)
    mean = (s.reshape(blk // 128, 128, 128) * inv[:, :, None]).reshape(blk, 128)
    h = jnp.maximum(
        jnp.dot(mean, u1_ref[...], preferred_element_type=jnp.float32) + ub1_ref[...],
        0.0,
    )
    o = jnp.dot(h, u2_ref[...], preferred_element_type=jnp.float32) + ub2_ref[...]
    nrm = jnp.maximum(jnp.sqrt(jnp.sum(o * o, axis=1, keepdims=True)), 1e-15)
    o_ref[...] = o * (jnp.tanh(nrm) / nrm)


def _update(acc, cnt2, U1, ub1, U2, ub2, blk):
    _, np_, d = acc.shape
    nw = cnt2.shape[0]
    grid = (np_ // blk,)
    return pl.pallas_call(
        functools.partial(_update_body, blk=blk),
        grid=grid,
        in_specs=[
            pl.BlockSpec((2, blk, d), lambda i: (0, i, 0)),
            pl.BlockSpec((nw, blk // 128, 128), lambda i: (0, i, 0)),
            pl.BlockSpec((d, d), lambda i: (0, 0)),
            pl.BlockSpec((1, d), lambda i: (0, 0)),
            pl.BlockSpec((d, d), lambda i: (0, 0)),
            pl.BlockSpec((1, d), lambda i: (0, 0)),
        ],
        out_specs=pl.BlockSpec((blk, d), lambda i: (i, 0)),
        out_shape=jax.ShapeDtypeStruct((np_, d), jnp.float32),
    )(acc, cnt2, U1, ub1.reshape(1, d), U2, ub2.reshape(1, d))


def _make_sc_scatter(np_, d, nw, nc, ns, ch, b):
    rows_per_tile = np_ // ns

    mesh = plsc.VectorSubcoreMesh(core_axis_name="c", subcore_axis_name="s")

    @functools.partial(
        pl.kernel,
        mesh=mesh,
        out_type=[
            jax.ShapeDtypeStruct((nc, np_, d), jnp.float32),
            jax.ShapeDtypeStruct((nc, np_), jnp.float32),
        ],
        scratch_types=[
            pltpu.VMEM((ch, b), jnp.int32),
            pltpu.VMEM((ch, b), jnp.int32),
            pltpu.VMEM((b, d), jnp.float32),
            pltpu.VMEM((b,), jnp.float32),
            pltpu.VMEM_SHARED((np_, d), jnp.float32),
            pltpu.VMEM_SHARED((np_,), jnp.float32),
            pltpu.SemaphoreType.DMA,
            pltpu.SemaphoreType.DMA,
        ],
    )
    def sc_kernel(
        m_hbm,
        row_hbm,
        col_hbm,
        acc_hbm,
        cnt_hbm,
        row_v,
        col_v,
        rows_v,
        ones_v,
        acc_sh,
        cnt_sh,
        gsem,
        csem,
    ):
        cid = lax.axis_index("c")
        sid = lax.axis_index("s")
        wid = sid * nc + cid

        def _fill(i, _):
            for j in range(d // 16):
                rows_v[i, pl.ds(j * 16, 16)] = jnp.zeros((16,), jnp.float32)
            return 0

        lax.fori_loop(0, b, _fill, 0)
        for j in range(b // 16):
            ones_v[pl.ds(j * 16, 16)] = jnp.ones((16,), jnp.float32)

        base = sid * rows_per_tile
        for k in range(rows_per_tile // b):
            pltpu.sync_copy(rows_v, acc_sh.at[pl.ds(base + k * b, b)])
        for k in range(rows_per_tile // d):
            pltpu.sync_copy(rows_v.at[0], cnt_sh.at[pl.ds(base + k * d, d)])
        plsc.subcore_barrier()

        pltpu.sync_copy(row_hbm.at[wid], row_v)
        pltpu.sync_copy(col_hbm.at[wid], col_v)

        def _chunk(c, _):
            pltpu.async_copy(m_hbm.at[col_v.at[c]], rows_v, gsem).wait()
            pltpu.async_copy(ones_v, cnt_sh.at[row_v.at[c]], csem, add=True)
            pltpu.sync_copy(rows_v, acc_sh.at[row_v.at[c]], add=True)
            return 0

        lax.fori_loop(0, ch, _chunk, 0)
        pltpu.make_async_copy(row_hbm.at[wid], row_v, csem).wait()
        plsc.subcore_barrier()

        pltpu.sync_copy(
            acc_sh.at[pl.ds(base, rows_per_tile)],
            acc_hbm.at[cid, pl.ds(base, rows_per_tile)],
        )
        pltpu.sync_copy(
            cnt_sh.at[pl.ds(base, rows_per_tile)],
            cnt_hbm.at[cid, pl.ds(base, rows_per_tile)],
        )

    return sc_kernel


def kernel(x, edge_index, W1, b1, W2, b2, U1, ub1, U2, ub2):
    n, d = x.shape
    e = edge_index.shape[1]

    info = plsc.get_sparse_core_info()
    nc, ns = info.num_cores, info.num_subcores
    nw = nc * ns
    b = 128
    quantum = nw * b
    ep = ((e + quantum - 1) // quantum) * quantum
    ch = ep // (nw * b)

    blk = 1024
    np_ = ((n + blk - 1) // blk) * blk

    x_pad = jnp.pad(x, ((0, np_ - n), (0, 0)))
    m = _node_messages(x_pad, W1, b1, W2, b2, blk)

    row = edge_index[0]
    col = edge_index[1]
    pad_e = ep - e
    row_p = jnp.concatenate([row, jnp.full((pad_e,), n, jnp.int32)]).reshape(nw, ch, b)
    col_p = jnp.concatenate([col, jnp.zeros((pad_e,), jnp.int32)]).reshape(nw, ch, b)

    acc, cnt = _make_sc_scatter(np_, d, nw, nc, ns, ch, b)(m, row_p, col_p)

    cnt2 = cnt.reshape(nc, np_ // 128, 128)
    out = _update(acc, cnt2, U1, ub1, U2, ub2, blk)
    return out[:n]
